# baseline (speedup 1.0000x reference)
import functools

import jax
import jax.numpy as jnp
from jax import lax
from jax.experimental import pallas as pl
from jax.experimental.pallas import tpu as pltpu
from jax.experimental.pallas import tpu_sc as plsc

N = 10000
D = 128
E = 320000

NC = 2
NS = 16
NW = NC * NS
K = 128
KB = 2
CH = 80
EPW = CH * K
E_PAD = NW * EPW
NPAD = 10240
RPT = NPAD // NS
ZR = 32
NMACRO = CH // KB

_mesh = plsc.VectorSubcoreMesh(
    core_axis_name="c", subcore_axis_name="s", num_cores=NC, num_subcores=NS
)


def _make_deg_kernel(width):
    @functools.partial(
        pl.kernel,
        out_type=jax.ShapeDtypeStruct((NC, NPAD, width), jnp.float32),
        mesh=_mesh,
        scratch_types=[
            pltpu.VMEM((KB, K), jnp.int32),
            pltpu.VMEM((K, width), jnp.float32),
            pltpu.VMEM((ZR, width), jnp.float32),
            pltpu.VMEM_SHARED((NPAD, width), jnp.float32),
            pltpu.SemaphoreType.DMA,
        ],
    )
    def _deg_kernel(dst_hbm, out_hbm, dst_v, ones_v, zero_v, acc, sem):
        cid = lax.axis_index("c")
        sid = lax.axis_index("s")
        wid = sid * NC + cid
        for i in range(K):
            for j in range(width // 16):
                ones_v[i, pl.ds(j * 16, 16)] = jnp.ones((16,), jnp.float32)
        for i in range(ZR):
            for j in range(width // 16):
                zero_v[i, pl.ds(j * 16, 16)] = jnp.zeros((16,), jnp.float32)
        base = sid * RPT
        for i in range(RPT // ZR):
            pltpu.sync_copy(zero_v, acc.at[pl.ds(base + i * ZR, ZR)])
        plsc.subcore_barrier()
        row0 = wid * CH

        @pl.loop(0, NMACRO)
        def _(m):
            pltpu.sync_copy(dst_hbm.at[pl.ds(row0 + m * KB, KB)], dst_v)
            hs = [
                pltpu.async_copy(ones_v, acc.at[dst_v.at[j]], sem, add=True)
                for j in range(KB)
            ]
            for h in hs:
                h.wait()

        plsc.subcore_barrier()
        pltpu.sync_copy(
            acc.at[pl.ds(base, RPT)], out_hbm.at[cid, pl.ds(base, RPT)]
        )

    return _deg_kernel


WDEG = 128
_deg_kernel = _make_deg_kernel(WDEG)


SEG = 32
SEGS0 = 5
SEGS1 = 0
ROWS1 = NS * SEGS0 * SEG


@functools.partial(
    pl.kernel,
    out_type=jax.ShapeDtypeStruct((NC, NPAD, D), jnp.float32),
    mesh=_mesh,
    scratch_types=[
        pltpu.VMEM((SEG, K), jnp.int32),
        pltpu.VMEM((SEG, K), jnp.int32),
        pltpu.VMEM((K, D), jnp.float32),
        pltpu.VMEM((K, D), jnp.float32),
        pltpu.VMEM_SHARED((NPAD, D), jnp.float32),
        pltpu.SemaphoreType.DMA,
        pltpu.SemaphoreType.DMA,
        pltpu.SemaphoreType.DMA,
        pltpu.SemaphoreType.DMA,
        pltpu.SemaphoreType.DMA,
    ],
)
def _agg_kernel(u_hbm, src_hbm, dst_hbm, out_hbm, src_t, dst_t,
                rows0, rows1, acc, gsem0, gsem1, ssem0, ssem1, zsem):
    cid = lax.axis_index("c")
    sid = lax.axis_index("s")
    rows = (rows0, rows1)
    gsem = (gsem0, gsem1)
    ssem = (ssem0, ssem1)
    for i in range(K):
        for j in range(D // 16):
            rows0[i, pl.ds(j * 16, 16)] = jnp.zeros((16,), jnp.float32)
    base = sid * RPT
    zh = [
        pltpu.async_copy(rows0, acc.at[pl.ds(base + i * K, K)], zsem)
        for i in range(RPT // K)
    ]
    for h in zh:
        h.wait()
    plsc.subcore_barrier()

    def pipeline(nseg, row_base):
        for hh in range(nseg):
            hbase = row_base + hh * SEG
            pltpu.sync_copy(src_hbm.at[pl.ds(hbase, SEG)], src_t)
            pltpu.sync_copy(dst_hbm.at[pl.ds(hbase, SEG)], dst_t)
            pend_g = {}
            pend_s = {}
            pend_g[0] = pltpu.async_copy(
                u_hbm.at[src_t.at[0]], rows[0], gsem[0]
            )
            pend_g[1] = pltpu.async_copy(
                u_hbm.at[src_t.at[1]], rows[1], gsem[1]
            )
            for m in range(SEG):
                b = m & 1
                pend_g[m].wait()
                pend_s[m] = pltpu.async_copy(
                    rows[b], acc.at[dst_t.at[m]], ssem[b], add=True
                )
                if m + 2 < SEG:
                    pend_s[m].wait()
                    pend_g[m + 2] = pltpu.async_copy(
                        u_hbm.at[src_t.at[m + 2]], rows[b], gsem[b]
                    )
            pend_s[SEG - 2].wait()
            pend_s[SEG - 1].wait()

    @pl.when(cid == 0)
    def _():
        pipeline(SEGS0, sid * (SEGS0 * SEG))

    if SEGS1 > 0:
        @pl.when(cid == 1)
        def _():
            pipeline(SEGS1, ROWS1 + sid * (SEGS1 * SEG))

    plsc.subcore_barrier()
    pltpu.sync_copy(acc.at[pl.ds(base, RPT)], out_hbm.at[cid, pl.ds(base, RPT)])


BM = 640


def _prep_body(d0_ref, d1_ref, x_ref, w_ref, dis_ref, u_ref):
    deg = 1.0 + d0_ref[:, 0:1] + d1_ref[:, 0:1]
    dis = lax.rsqrt(deg)
    dis_ref[...] = jnp.broadcast_to(dis, (BM, D))
    h = jnp.dot(x_ref[...], w_ref[...],
                preferred_element_type=jnp.float32,
                precision=lax.Precision.HIGHEST)
    u_ref[...] = dis * h


_prep = pl.pallas_call(
    _prep_body,
    grid=(NPAD // BM,),
    in_specs=[
        pl.BlockSpec((BM, WDEG), lambda i: (i, 0)),
        pl.BlockSpec((BM, WDEG), lambda i: (i, 0)),
        pl.BlockSpec((BM, D), lambda i: (i, 0)),
        pl.BlockSpec((D, D), lambda i: (0, 0)),
    ],
    out_specs=[
        pl.BlockSpec((BM, D), lambda i: (i, 0)),
        pl.BlockSpec((BM, D), lambda i: (i, 0)),
    ],
    out_shape=[
        jax.ShapeDtypeStruct((NPAD, D), jnp.float32),
        jax.ShapeDtypeStruct((NPAD, D), jnp.float32),
    ],
)


def _layer_body(p0_ref, p1_ref, u_ref, dis_ref, b_ref, w_ref, out_ref):
    dis = dis_ref[...]
    xb = jnp.maximum(
        dis * (p0_ref[...] + p1_ref[...] + u_ref[...]) + b_ref[...], 0.0
    )
    out_ref[...] = dis * jnp.dot(
        xb, w_ref[...],
        preferred_element_type=jnp.float32,
        precision=lax.Precision.HIGHEST,
    )


_layer = pl.pallas_call(
    _layer_body,
    grid=(NPAD // BM,),
    in_specs=[
        pl.BlockSpec((BM, D), lambda i: (i, 0)),
        pl.BlockSpec((BM, D), lambda i: (i, 0)),
        pl.BlockSpec((BM, D), lambda i: (i, 0)),
        pl.BlockSpec((BM, D), lambda i: (i, 0)),
        pl.BlockSpec((1, D), lambda i: (0, 0)),
        pl.BlockSpec((D, D), lambda i: (0, 0)),
    ],
    out_specs=pl.BlockSpec((BM, D), lambda i: (i, 0)),
    out_shape=jax.ShapeDtypeStruct((NPAD, D), jnp.float32),
)


def _final_body(p0_ref, p1_ref, u_ref, dis_ref, b_ref, out_ref):
    out_ref[...] = (
        dis_ref[...] * (p0_ref[...] + p1_ref[...] + u_ref[...]) + b_ref[...]
    )


_final = pl.pallas_call(
    _final_body,
    grid=(NPAD // BM,),
    in_specs=[
        pl.BlockSpec((BM, D), lambda i: (i, 0)),
        pl.BlockSpec((BM, D), lambda i: (i, 0)),
        pl.BlockSpec((BM, D), lambda i: (i, 0)),
        pl.BlockSpec((BM, D), lambda i: (i, 0)),
        pl.BlockSpec((1, D), lambda i: (0, 0)),
    ],
    out_specs=pl.BlockSpec((BM, D), lambda i: (i, 0)),
    out_shape=jax.ShapeDtypeStruct((NPAD, D), jnp.float32),
)


def kernel(x, edge_index, W1, b1, W2, b2, W3, b3, W4, b4):
    src = edge_index[0].astype(jnp.int32)
    dst = edge_index[1].astype(jnp.int32)
    pad = E_PAD - E
    src2 = jnp.concatenate([src, jnp.zeros((pad,), jnp.int32)]).reshape(
        E_PAD // K, K
    )
    dst_pad = N + (jnp.arange(pad, dtype=jnp.int32) % (NPAD - N))
    dst2 = jnp.concatenate([dst, dst_pad]).reshape(E_PAD // K, K)
    xp = jnp.pad(x, ((0, NPAD - N), (0, 0)))

    degp = _deg_kernel(dst2)
    dis, u = _prep(degp[0], degp[1], xp, W1)

    for (b_prev, w_next) in ((b1, W2), (b2, W3), (b3, W4)):
        p = _agg_kernel(u, src2, dst2)
        u = _layer(p[0], p[1], u, dis, b_prev.reshape(1, D), w_next)

    p = _agg_kernel(u, src2, dst2)
    out = _final(p[0], p[1], u, dis, b4.reshape(1, D))
    return out[:N]


# speedup vs baseline: 3.5470x; 3.5470x over previous
import functools

import jax
import jax.numpy as jnp
from jax import lax
from jax.experimental import pallas as pl
from jax.experimental.pallas import tpu as pltpu
from jax.experimental.pallas import tpu_sc as plsc

N = 10000
D = 128
E = 320000

NC = 2
NS = 16
NW = NC * NS
K = 128
KB = 2
CH = 80
EPW = CH * K
E_PAD = NW * EPW
NPAD = 10240
RPT = NPAD // NS
ZR = 32
NMACRO = CH // KB

_mesh = plsc.VectorSubcoreMesh(
    core_axis_name="c", subcore_axis_name="s", num_cores=NC, num_subcores=NS
)


def _make_deg_kernel(width):
    @functools.partial(
        pl.kernel,
        out_type=jax.ShapeDtypeStruct((NC, NPAD, width), jnp.float32),
        mesh=_mesh,
        scratch_types=[
            pltpu.VMEM((KB, K), jnp.int32),
            pltpu.VMEM((K, width), jnp.float32),
            pltpu.VMEM((ZR, width), jnp.float32),
            pltpu.VMEM_SHARED((NPAD, width), jnp.float32),
            pltpu.SemaphoreType.DMA,
        ],
    )
    def _deg_kernel(dst_hbm, out_hbm, dst_v, ones_v, zero_v, acc, sem):
        cid = lax.axis_index("c")
        sid = lax.axis_index("s")
        wid = sid * NC + cid
        for i in range(K):
            for j in range(width // 16):
                ones_v[i, pl.ds(j * 16, 16)] = jnp.ones((16,), jnp.float32)
        for i in range(ZR):
            for j in range(width // 16):
                zero_v[i, pl.ds(j * 16, 16)] = jnp.zeros((16,), jnp.float32)
        base = sid * RPT
        for i in range(RPT // ZR):
            pltpu.sync_copy(zero_v, acc.at[pl.ds(base + i * ZR, ZR)])
        plsc.subcore_barrier()
        row0 = wid * CH

        @pl.loop(0, NMACRO)
        def _(m):
            pltpu.sync_copy(dst_hbm.at[pl.ds(row0 + m * KB, KB)], dst_v)
            hs = [
                pltpu.async_copy(ones_v, acc.at[dst_v.at[j]], sem, add=True)
                for j in range(KB)
            ]
            for h in hs:
                h.wait()

        plsc.subcore_barrier()
        pltpu.sync_copy(
            acc.at[pl.ds(base, RPT)], out_hbm.at[cid, pl.ds(base, RPT)]
        )

    return _deg_kernel


WDEG = 128
_deg_kernel = _make_deg_kernel(WDEG)


SEG = 40
SEGS = 2


@functools.partial(
    pl.kernel,
    out_type=jax.ShapeDtypeStruct((NC, NPAD, D), jnp.float32),
    mesh=_mesh,
    scratch_types=[
        pltpu.VMEM((SEG, K), jnp.int32),
        pltpu.VMEM((SEG, K), jnp.int32),
        pltpu.VMEM((K, D), jnp.float32),
        pltpu.VMEM((K, D), jnp.float32),
        pltpu.VMEM_SHARED((NPAD, D), jnp.float32),
        pltpu.SemaphoreType.DMA,
        pltpu.SemaphoreType.DMA,
        pltpu.SemaphoreType.DMA,
        pltpu.SemaphoreType.DMA,
        pltpu.SemaphoreType.DMA,
    ],
)
def _agg_kernel(u_hbm, src_hbm, dst_hbm, out_hbm, src_t, dst_t,
                rows0, rows1, acc, gsem0, gsem1, ssem0, ssem1, zsem):
    cid = lax.axis_index("c")
    sid = lax.axis_index("s")
    rows = (rows0, rows1)
    gsem = (gsem0, gsem1)
    ssem = (ssem0, ssem1)
    for i in range(K):
        for j in range(D // 16):
            rows0[i, pl.ds(j * 16, 16)] = jnp.zeros((16,), jnp.float32)
    base = sid * RPT
    zh = [
        pltpu.async_copy(rows0, acc.at[pl.ds(base + i * K, K)], zsem)
        for i in range(RPT // K)
    ]
    for h in zh:
        h.wait()
    plsc.subcore_barrier()

    def pipeline(nseg, row_base):
        for hh in range(nseg):
            hbase = row_base + hh * SEG
            pltpu.sync_copy(src_hbm.at[pl.ds(hbase, SEG)], src_t)
            pltpu.sync_copy(dst_hbm.at[pl.ds(hbase, SEG)], dst_t)
            pend_g = {}
            pend_s = {}
            pend_g[0] = pltpu.async_copy(
                u_hbm.at[src_t.at[0]], rows[0], gsem[0]
            )
            pend_g[1] = pltpu.async_copy(
                u_hbm.at[src_t.at[1]], rows[1], gsem[1]
            )
            for m in range(SEG):
                b = m & 1
                pend_g[m].wait()
                pend_s[m] = pltpu.async_copy(
                    rows[b], acc.at[dst_t.at[m]], ssem[b], add=True
                )
                if m + 2 < SEG:
                    pend_s[m].wait()
                    pend_g[m + 2] = pltpu.async_copy(
                        u_hbm.at[src_t.at[m + 2]], rows[b], gsem[b]
                    )
            pend_s[SEG - 2].wait()
            pend_s[SEG - 1].wait()

    pipeline(SEGS, (cid * NS + sid) * (SEGS * SEG))

    plsc.subcore_barrier()
    pltpu.sync_copy(acc.at[pl.ds(base, RPT)], out_hbm.at[cid, pl.ds(base, RPT)])


BM = 640


def _prep_body(d0_ref, d1_ref, x_ref, w_ref, dis_ref, u_ref):
    deg = 1.0 + d0_ref[:, 0:1] + d1_ref[:, 0:1]
    dis = lax.rsqrt(deg)
    dis_ref[...] = jnp.broadcast_to(dis, (BM, D))
    h = jnp.dot(x_ref[...], w_ref[...],
                preferred_element_type=jnp.float32,
                precision=lax.Precision.HIGHEST)
    u_ref[...] = dis * h


_prep = pl.pallas_call(
    _prep_body,
    grid=(NPAD // BM,),
    in_specs=[
        pl.BlockSpec((BM, WDEG), lambda i: (i, 0)),
        pl.BlockSpec((BM, WDEG), lambda i: (i, 0)),
        pl.BlockSpec((BM, D), lambda i: (i, 0)),
        pl.BlockSpec((D, D), lambda i: (0, 0)),
    ],
    out_specs=[
        pl.BlockSpec((BM, D), lambda i: (i, 0)),
        pl.BlockSpec((BM, D), lambda i: (i, 0)),
    ],
    out_shape=[
        jax.ShapeDtypeStruct((NPAD, D), jnp.float32),
        jax.ShapeDtypeStruct((NPAD, D), jnp.float32),
    ],
)


def _layer_body(p0_ref, p1_ref, u_ref, dis_ref, b_ref, w_ref, out_ref):
    dis = dis_ref[...]
    xb = jnp.maximum(
        dis * (p0_ref[...] + p1_ref[...] + u_ref[...]) + b_ref[...], 0.0
    )
    out_ref[...] = dis * jnp.dot(
        xb, w_ref[...],
        preferred_element_type=jnp.float32,
        precision=lax.Precision.HIGHEST,
    )


_layer = pl.pallas_call(
    _layer_body,
    grid=(NPAD // BM,),
    in_specs=[
        pl.BlockSpec((BM, D), lambda i: (i, 0)),
        pl.BlockSpec((BM, D), lambda i: (i, 0)),
        pl.BlockSpec((BM, D), lambda i: (i, 0)),
        pl.BlockSpec((BM, D), lambda i: (i, 0)),
        pl.BlockSpec((1, D), lambda i: (0, 0)),
        pl.BlockSpec((D, D), lambda i: (0, 0)),
    ],
    out_specs=pl.BlockSpec((BM, D), lambda i: (i, 0)),
    out_shape=jax.ShapeDtypeStruct((NPAD, D), jnp.float32),
)


def _final_body(p0_ref, p1_ref, u_ref, dis_ref, b_ref, out_ref):
    out_ref[...] = (
        dis_ref[...] * (p0_ref[...] + p1_ref[...] + u_ref[...]) + b_ref[...]
    )


_final = pl.pallas_call(
    _final_body,
    grid=(NPAD // BM,),
    in_specs=[
        pl.BlockSpec((BM, D), lambda i: (i, 0)),
        pl.BlockSpec((BM, D), lambda i: (i, 0)),
        pl.BlockSpec((BM, D), lambda i: (i, 0)),
        pl.BlockSpec((BM, D), lambda i: (i, 0)),
        pl.BlockSpec((1, D), lambda i: (0, 0)),
    ],
    out_specs=pl.BlockSpec((BM, D), lambda i: (i, 0)),
    out_shape=jax.ShapeDtypeStruct((NPAD, D), jnp.float32),
)


def kernel(x, edge_index, W1, b1, W2, b2, W3, b3, W4, b4):
    src = edge_index[0].astype(jnp.int32)
    dst = edge_index[1].astype(jnp.int32)
    pad = E_PAD - E
    src_pad = jnp.arange(pad, dtype=jnp.int32) % N
    src2 = jnp.concatenate([src, src_pad]).reshape(E_PAD // K, K)
    dst_pad = N + (jnp.arange(pad, dtype=jnp.int32) % (NPAD - N))
    dst2 = jnp.concatenate([dst, dst_pad]).reshape(E_PAD // K, K)
    xp = jnp.pad(x, ((0, NPAD - N), (0, 0)))

    degp = _deg_kernel(dst2)
    dis, u = _prep(degp[0], degp[1], xp, W1)

    for (b_prev, w_next) in ((b1, W2), (b2, W3), (b3, W4)):
        p = _agg_kernel(u, src2, dst2)
        u = _layer(p[0], p[1], u, dis, b_prev.reshape(1, D), w_next)

    p = _agg_kernel(u, src2, dst2)
    out = _final(p[0], p[1], u, dis, b4.reshape(1, D))
    return out[:N]


# speedup vs baseline: 3.7655x; 1.0616x over previous
import functools

import jax
import jax.numpy as jnp
from jax import lax
from jax.experimental import pallas as pl
from jax.experimental.pallas import tpu as pltpu
from jax.experimental.pallas import tpu_sc as plsc

N = 10000
D = 128
E = 320000

NC = 2
NS = 16
NW = NC * NS
K = 128
KB = 2
CH = 80
EPW = CH * K
E_PAD = NW * EPW
NPAD = 10240
RPT = NPAD // NS
ZR = 32
NMACRO = CH // KB

_mesh = plsc.VectorSubcoreMesh(
    core_axis_name="c", subcore_axis_name="s", num_cores=NC, num_subcores=NS
)


def _make_deg_kernel(width):
    @functools.partial(
        pl.kernel,
        out_type=jax.ShapeDtypeStruct((NC, NPAD, width), jnp.float32),
        mesh=_mesh,
        scratch_types=[
            pltpu.VMEM((KB, K), jnp.int32),
            pltpu.VMEM((K, width), jnp.float32),
            pltpu.VMEM((ZR, width), jnp.float32),
            pltpu.VMEM_SHARED((NPAD, width), jnp.float32),
            pltpu.SemaphoreType.DMA,
        ],
    )
    def _deg_kernel(dst_hbm, out_hbm, dst_v, ones_v, zero_v, acc, sem):
        cid = lax.axis_index("c")
        sid = lax.axis_index("s")
        wid = sid * NC + cid
        for i in range(K):
            for j in range(width // 16):
                ones_v[i, pl.ds(j * 16, 16)] = jnp.ones((16,), jnp.float32)
        for i in range(ZR):
            for j in range(width // 16):
                zero_v[i, pl.ds(j * 16, 16)] = jnp.zeros((16,), jnp.float32)
        base = sid * RPT
        for i in range(RPT // ZR):
            pltpu.sync_copy(zero_v, acc.at[pl.ds(base + i * ZR, ZR)])
        plsc.subcore_barrier()
        row0 = wid * CH

        @pl.loop(0, NMACRO)
        def _(m):
            pltpu.sync_copy(dst_hbm.at[pl.ds(row0 + m * KB, KB)], dst_v)
            hs = [
                pltpu.async_copy(ones_v, acc.at[dst_v.at[j]], sem, add=True)
                for j in range(KB)
            ]
            for h in hs:
                h.wait()

        plsc.subcore_barrier()
        pltpu.sync_copy(
            acc.at[pl.ds(base, RPT)], out_hbm.at[cid, pl.ds(base, RPT)]
        )

    return _deg_kernel


WDEG = 128
_deg_kernel = _make_deg_kernel(WDEG)


SEG = 40
SEGS = 2


@functools.partial(
    pl.kernel,
    out_type=jax.ShapeDtypeStruct((NC, NPAD, D), jnp.float32),
    mesh=_mesh,
    scratch_types=[
        pltpu.VMEM((SEG, K), jnp.int32),
        pltpu.VMEM((SEG, K), jnp.int32),
        pltpu.VMEM((K, D), jnp.float32),
        pltpu.VMEM((K, D), jnp.float32),
        pltpu.VMEM_SHARED((NPAD, D), jnp.float32),
        pltpu.SemaphoreType.DMA,
        pltpu.SemaphoreType.DMA,
        pltpu.SemaphoreType.DMA,
        pltpu.SemaphoreType.DMA,
        pltpu.SemaphoreType.DMA,
    ],
)
def _agg_kernel(u_hbm, src_hbm, dst_hbm, out_hbm, src_t, dst_t,
                rows0, rows1, acc, gsem0, gsem1, ssem0, ssem1, zsem):
    cid = lax.axis_index("c")
    sid = lax.axis_index("s")
    rows = (rows0, rows1)
    gsem = (gsem0, gsem1)
    ssem = (ssem0, ssem1)
    for i in range(K):
        for j in range(D // 16):
            rows0[i, pl.ds(j * 16, 16)] = jnp.zeros((16,), jnp.float32)
    base = sid * RPT
    zh = [
        pltpu.async_copy(rows0, acc.at[pl.ds(base + i * K, K)], zsem)
        for i in range(RPT // K)
    ]
    for h in zh:
        h.wait()
    plsc.subcore_barrier()

    def pipeline(nseg, row_base):
        for hh in range(nseg):
            hbase = row_base + hh * SEG
            pltpu.sync_copy(src_hbm.at[pl.ds(hbase, SEG)], src_t)
            pltpu.sync_copy(dst_hbm.at[pl.ds(hbase, SEG)], dst_t)
            pend_g = {}
            pend_s = {}
            pend_g[0] = pltpu.async_copy(
                u_hbm.at[src_t.at[0]], rows[0], gsem[0]
            )
            pend_g[1] = pltpu.async_copy(
                u_hbm.at[src_t.at[1]], rows[1], gsem[1]
            )
            for m in range(SEG):
                b = m & 1
                pend_g[m].wait()
                pend_s[m] = pltpu.async_copy(
                    rows[b], acc.at[dst_t.at[m]], ssem[b], add=True
                )
                if m + 2 < SEG:
                    pend_s[m].wait()
                    pend_g[m + 2] = pltpu.async_copy(
                        u_hbm.at[src_t.at[m + 2]], rows[b], gsem[b]
                    )
            pend_s[SEG - 2].wait()
            pend_s[SEG - 1].wait()

    pipeline(SEGS, (cid * NS + sid) * (SEGS * SEG))

    plsc.subcore_barrier()
    pltpu.sync_copy(acc.at[pl.ds(base, RPT)], out_hbm.at[cid, pl.ds(base, RPT)])


BM = 640


def _prep_body(dp0_ref, dp1_ref, x_ref, w_ref, dis_ref, u_ref):
    deg = 1.0 + dp0_ref[0, :, 0:1] + dp1_ref[0, :, 0:1]
    dis = lax.rsqrt(deg)
    dis_ref[...] = jnp.broadcast_to(dis, (BM, D))
    h = jnp.dot(x_ref[...], w_ref[...],
                preferred_element_type=jnp.float32,
                precision=lax.Precision.HIGHEST)
    u_ref[...] = dis * h


_prep = pl.pallas_call(
    _prep_body,
    grid=(NPAD // BM,),
    in_specs=[
        pl.BlockSpec((1, BM, D), lambda i: (0, i, 0)),
        pl.BlockSpec((1, BM, D), lambda i: (1, i, 0)),
        pl.BlockSpec((BM, D), lambda i: (i, 0)),
        pl.BlockSpec((D, D), lambda i: (0, 0)),
    ],
    out_specs=[
        pl.BlockSpec((BM, D), lambda i: (i, 0)),
        pl.BlockSpec((BM, D), lambda i: (i, 0)),
    ],
    out_shape=[
        jax.ShapeDtypeStruct((NPAD, D), jnp.float32),
        jax.ShapeDtypeStruct((NPAD, D), jnp.float32),
    ],
)


def _layer_body(p0_ref, p1_ref, u_ref, dis_ref, b_ref, w_ref, out_ref):
    dis = dis_ref[...]
    xb = jnp.maximum(
        dis * (p0_ref[0] + p1_ref[0] + u_ref[...]) + b_ref[...], 0.0
    )
    out_ref[...] = dis * jnp.dot(
        xb, w_ref[...],
        preferred_element_type=jnp.float32,
        precision=lax.Precision.HIGHEST,
    )


_layer = pl.pallas_call(
    _layer_body,
    grid=(NPAD // BM,),
    in_specs=[
        pl.BlockSpec((1, BM, D), lambda i: (0, i, 0)),
        pl.BlockSpec((1, BM, D), lambda i: (1, i, 0)),
        pl.BlockSpec((BM, D), lambda i: (i, 0)),
        pl.BlockSpec((BM, D), lambda i: (i, 0)),
        pl.BlockSpec((1, D), lambda i: (0, 0)),
        pl.BlockSpec((D, D), lambda i: (0, 0)),
    ],
    out_specs=pl.BlockSpec((BM, D), lambda i: (i, 0)),
    out_shape=jax.ShapeDtypeStruct((NPAD, D), jnp.float32),
)


BMF = 1000


def _final_body(p0_ref, p1_ref, u_ref, dis_ref, b_ref, out_ref):
    out_ref[...] = (
        dis_ref[...] * (p0_ref[0] + p1_ref[0] + u_ref[...]) + b_ref[...]
    )


_final = pl.pallas_call(
    _final_body,
    grid=(N // BMF,),
    in_specs=[
        pl.BlockSpec((1, BMF, D), lambda i: (0, i, 0)),
        pl.BlockSpec((1, BMF, D), lambda i: (1, i, 0)),
        pl.BlockSpec((BMF, D), lambda i: (i, 0)),
        pl.BlockSpec((BMF, D), lambda i: (i, 0)),
        pl.BlockSpec((1, D), lambda i: (0, 0)),
    ],
    out_specs=pl.BlockSpec((BMF, D), lambda i: (i, 0)),
    out_shape=jax.ShapeDtypeStruct((N, D), jnp.float32),
)


def kernel(x, edge_index, W1, b1, W2, b2, W3, b3, W4, b4):
    src = edge_index[0].astype(jnp.int32)
    dst = edge_index[1].astype(jnp.int32)
    pad = E_PAD - E
    src_pad = jnp.arange(pad, dtype=jnp.int32) % N
    src2 = jnp.concatenate([src, src_pad]).reshape(E_PAD // K, K)
    dst_pad = N + (jnp.arange(pad, dtype=jnp.int32) % (NPAD - N))
    dst2 = jnp.concatenate([dst, dst_pad]).reshape(E_PAD // K, K)
    xp = jnp.pad(x, ((0, NPAD - N), (0, 0)))

    degp = _deg_kernel(dst2)
    dis, u = _prep(degp, degp, xp, W1)

    for (b_prev, w_next) in ((b1, W2), (b2, W3), (b3, W4)):
        p = _agg_kernel(u, src2, dst2)
        u = _layer(p, p, u, dis, b_prev.reshape(1, D), w_next)

    p = _agg_kernel(u, src2, dst2)
    return _final(p, p, u, dis, b4.reshape(1, D))


# speedup vs baseline: 4.2165x; 1.1198x over previous
import functools

import jax
import jax.numpy as jnp
from jax import lax
from jax.experimental import pallas as pl
from jax.experimental.pallas import tpu as pltpu
from jax.experimental.pallas import tpu_sc as plsc

N = 10000
D = 128
E = 320000

NC = 2
NS = 16
NW = NC * NS
K = 128
KB = 2
CH = 80
EPW = CH * K
E_PAD = NW * EPW
NPAD = 10240
RPT = NPAD // NS
ZR = 32
NMACRO = CH // KB

_mesh = plsc.VectorSubcoreMesh(
    core_axis_name="c", subcore_axis_name="s", num_cores=NC, num_subcores=NS
)


@functools.partial(
    pl.kernel,
    out_type=jax.ShapeDtypeStruct((NC, NPAD), jnp.float32),
    mesh=_mesh,
    compiler_params=pltpu.CompilerParams(needs_layout_passes=False),
    scratch_types=[
        pltpu.VMEM((40, K), jnp.int32),
        pltpu.VMEM((NPAD,), jnp.float32),
        pltpu.VMEM((NS, NPAD // NS), jnp.float32),
        pltpu.VMEM((NPAD // NS,), jnp.float32),
        pltpu.VMEM_SHARED((NS, NPAD), jnp.float32),
    ],
)
def _deg_kernel(dst_hbm, out_hbm, dst_t, local, redbuf, sumbuf, slots):
    cid = lax.axis_index("c")
    sid = lax.axis_index("s")
    wid = cid * NS + sid

    @pl.loop(0, NPAD // 16)
    def _(i):
        local[pl.ds(i * 16, 16)] = jnp.zeros((16,), jnp.float32)

    ones = jnp.ones((16,), jnp.float32)
    row0 = wid * CH
    for seg in range(2):
        pltpu.sync_copy(dst_hbm.at[pl.ds(row0 + seg * 40, 40)], dst_t)

        @pl.loop(0, 40)
        def _(r):
            for j in range(K // 16):
                iv = dst_t[r, pl.ds(j * 16, 16)]
                plsc.addupdate_scatter(local, [iv], ones)

    pltpu.sync_copy(local, slots.at[sid])
    plsc.subcore_barrier()
    base = sid * RPT
    pltpu.sync_copy(slots.at[:, pl.ds(base, RPT)], redbuf)

    @pl.loop(0, RPT // 16)
    def _(g):
        v = redbuf[0, pl.ds(g * 16, 16)]
        for t in range(1, NS):
            v = v + redbuf[t, pl.ds(g * 16, 16)]
        sumbuf[pl.ds(g * 16, 16)] = v

    pltpu.sync_copy(sumbuf, out_hbm.at[cid, pl.ds(base, RPT)])


SEG = 40
SEGS = 2


@functools.partial(
    pl.kernel,
    out_type=jax.ShapeDtypeStruct((NC, NPAD, D), jnp.float32),
    mesh=_mesh,
    scratch_types=[
        pltpu.VMEM((SEG, K), jnp.int32),
        pltpu.VMEM((SEG, K), jnp.int32),
        pltpu.VMEM((K, D), jnp.float32),
        pltpu.VMEM((K, D), jnp.float32),
        pltpu.VMEM_SHARED((NPAD, D), jnp.float32),
        pltpu.SemaphoreType.DMA,
        pltpu.SemaphoreType.DMA,
        pltpu.SemaphoreType.DMA,
        pltpu.SemaphoreType.DMA,
        pltpu.SemaphoreType.DMA,
    ],
)
def _agg_kernel(u_hbm, src_hbm, dst_hbm, out_hbm, src_t, dst_t,
                rows0, rows1, acc, gsem0, gsem1, ssem0, ssem1, zsem):
    cid = lax.axis_index("c")
    sid = lax.axis_index("s")
    rows = (rows0, rows1)
    gsem = (gsem0, gsem1)
    ssem = (ssem0, ssem1)
    for i in range(K):
        for j in range(D // 16):
            rows0[i, pl.ds(j * 16, 16)] = jnp.zeros((16,), jnp.float32)
    base = sid * RPT
    zh = [
        pltpu.async_copy(rows0, acc.at[pl.ds(base + i * K, K)], zsem)
        for i in range(RPT // K)
    ]
    for h in zh:
        h.wait()
    plsc.subcore_barrier()

    def pipeline(nseg, row_base):
        for hh in range(nseg):
            hbase = row_base + hh * SEG
            pltpu.sync_copy(src_hbm.at[pl.ds(hbase, SEG)], src_t)
            pltpu.sync_copy(dst_hbm.at[pl.ds(hbase, SEG)], dst_t)
            pend_g = {}
            pend_s = {}
            pend_g[0] = pltpu.async_copy(
                u_hbm.at[src_t.at[0]], rows[0], gsem[0]
            )
            pend_g[1] = pltpu.async_copy(
                u_hbm.at[src_t.at[1]], rows[1], gsem[1]
            )
            for m in range(SEG):
                b = m & 1
                pend_g[m].wait()
                pend_s[m] = pltpu.async_copy(
                    rows[b], acc.at[dst_t.at[m]], ssem[b], add=True
                )
                if m + 2 < SEG:
                    pend_s[m].wait()
                    pend_g[m + 2] = pltpu.async_copy(
                        u_hbm.at[src_t.at[m + 2]], rows[b], gsem[b]
                    )
            pend_s[SEG - 2].wait()
            pend_s[SEG - 1].wait()

    pipeline(SEGS, (cid * NS + sid) * (SEGS * SEG))

    plsc.subcore_barrier()
    pltpu.sync_copy(acc.at[pl.ds(base, RPT)], out_hbm.at[cid, pl.ds(base, RPT)])


BM = 640


def _prep_body(dp0_ref, dp1_ref, x_ref, w_ref, dis_ref, u_ref):
    deg = 1.0 + dp0_ref[0] + dp1_ref[0]
    dis = lax.rsqrt(deg)
    dis_ref[...] = jnp.broadcast_to(dis, (BM, D))
    h = jnp.dot(x_ref[...], w_ref[...],
                preferred_element_type=jnp.float32,
                precision=lax.Precision.HIGHEST)
    u_ref[...] = dis * h


_prep = pl.pallas_call(
    _prep_body,
    grid=(NPAD // BM,),
    in_specs=[
        pl.BlockSpec((1, BM, 1), lambda i: (0, i, 0)),
        pl.BlockSpec((1, BM, 1), lambda i: (1, i, 0)),
        pl.BlockSpec((BM, D), lambda i: (i, 0)),
        pl.BlockSpec((D, D), lambda i: (0, 0)),
    ],
    out_specs=[
        pl.BlockSpec((BM, D), lambda i: (i, 0)),
        pl.BlockSpec((BM, D), lambda i: (i, 0)),
    ],
    out_shape=[
        jax.ShapeDtypeStruct((NPAD, D), jnp.float32),
        jax.ShapeDtypeStruct((NPAD, D), jnp.float32),
    ],
)


def _layer_body(p0_ref, p1_ref, u_ref, dis_ref, b_ref, w_ref, out_ref):
    dis = dis_ref[...]
    xb = jnp.maximum(
        dis * (p0_ref[0] + p1_ref[0] + u_ref[...]) + b_ref[...], 0.0
    )
    out_ref[...] = dis * jnp.dot(
        xb, w_ref[...],
        preferred_element_type=jnp.float32,
        precision=lax.Precision.HIGHEST,
    )


_layer = pl.pallas_call(
    _layer_body,
    grid=(NPAD // BM,),
    in_specs=[
        pl.BlockSpec((1, BM, D), lambda i: (0, i, 0)),
        pl.BlockSpec((1, BM, D), lambda i: (1, i, 0)),
        pl.BlockSpec((BM, D), lambda i: (i, 0)),
        pl.BlockSpec((BM, D), lambda i: (i, 0)),
        pl.BlockSpec((1, D), lambda i: (0, 0)),
        pl.BlockSpec((D, D), lambda i: (0, 0)),
    ],
    out_specs=pl.BlockSpec((BM, D), lambda i: (i, 0)),
    out_shape=jax.ShapeDtypeStruct((NPAD, D), jnp.float32),
)


BMF = 1000


def _final_body(p0_ref, p1_ref, u_ref, dis_ref, b_ref, out_ref):
    out_ref[...] = (
        dis_ref[...] * (p0_ref[0] + p1_ref[0] + u_ref[...]) + b_ref[...]
    )


_final = pl.pallas_call(
    _final_body,
    grid=(N // BMF,),
    in_specs=[
        pl.BlockSpec((1, BMF, D), lambda i: (0, i, 0)),
        pl.BlockSpec((1, BMF, D), lambda i: (1, i, 0)),
        pl.BlockSpec((BMF, D), lambda i: (i, 0)),
        pl.BlockSpec((BMF, D), lambda i: (i, 0)),
        pl.BlockSpec((1, D), lambda i: (0, 0)),
    ],
    out_specs=pl.BlockSpec((BMF, D), lambda i: (i, 0)),
    out_shape=jax.ShapeDtypeStruct((N, D), jnp.float32),
)


def kernel(x, edge_index, W1, b1, W2, b2, W3, b3, W4, b4):
    src = edge_index[0].astype(jnp.int32)
    dst = edge_index[1].astype(jnp.int32)
    pad = E_PAD - E
    src_pad = jnp.arange(pad, dtype=jnp.int32) % N
    src2 = jnp.concatenate([src, src_pad]).reshape(E_PAD // K, K)
    dst_pad = N + (jnp.arange(pad, dtype=jnp.int32) % (NPAD - N))
    dst2 = jnp.concatenate([dst, dst_pad]).reshape(E_PAD // K, K)
    xp = jnp.pad(x, ((0, NPAD - N), (0, 0)))

    degp = _deg_kernel(dst2).reshape(NC, NPAD, 1)
    dis, u = _prep(degp, degp, xp, W1)

    for (b_prev, w_next) in ((b1, W2), (b2, W3), (b3, W4)):
        p = _agg_kernel(u, src2, dst2)
        u = _layer(p, p, u, dis, b_prev.reshape(1, D), w_next)

    p = _agg_kernel(u, src2, dst2)
    return _final(p, p, u, dis, b4.reshape(1, D))


# speedup vs baseline: 4.3291x; 1.0267x over previous
import functools

import jax
import jax.numpy as jnp
from jax import lax
from jax.experimental import pallas as pl
from jax.experimental.pallas import tpu as pltpu
from jax.experimental.pallas import tpu_sc as plsc

N = 10000
D = 128
E = 320000

NC = 2
NS = 16
NW = NC * NS
KE = 125
CR = E // KE
CH = CR // NW
NPAD = 10240
RPT = NPAD // NS
DROWS = 2560
DCH = DROWS // NW

_mesh = plsc.VectorSubcoreMesh(
    core_axis_name="c", subcore_axis_name="s", num_cores=NC, num_subcores=NS
)


@functools.partial(
    pl.kernel,
    out_type=jax.ShapeDtypeStruct((NC, NPAD), jnp.float32),
    mesh=_mesh,
    compiler_params=pltpu.CompilerParams(needs_layout_passes=False),
    scratch_types=[
        pltpu.VMEM((40, 128), jnp.int32),
        pltpu.VMEM((NPAD,), jnp.float32),
        pltpu.VMEM((NS, NPAD // NS), jnp.float32),
        pltpu.VMEM((NPAD // NS,), jnp.float32),
        pltpu.VMEM_SHARED((NS, NPAD), jnp.float32),
    ],
)
def _deg_kernel(dst_hbm, out_hbm, dst_t, local, redbuf, sumbuf, slots):
    cid = lax.axis_index("c")
    sid = lax.axis_index("s")
    wid = cid * NS + sid

    @pl.loop(0, NPAD // 16)
    def _(i):
        local[pl.ds(i * 16, 16)] = jnp.zeros((16,), jnp.float32)

    ones = jnp.ones((16,), jnp.float32)
    row0 = wid * (DCH // 2) * 2
    for seg in range(2):
        pltpu.sync_copy(dst_hbm.at[pl.ds(row0 + seg * (DCH // 2), DCH // 2)],
                        dst_t.at[pl.ds(0, DCH // 2)])

        @pl.loop(0, DCH // 2)
        def _(r):
            for j in range(8):
                iv = dst_t[r, pl.ds(j * 16, 16)]
                plsc.addupdate_scatter(local, [iv], ones)

    pltpu.sync_copy(local, slots.at[sid])
    plsc.subcore_barrier()
    base = sid * RPT
    pltpu.sync_copy(slots.at[:, pl.ds(base, RPT)], redbuf)

    @pl.loop(0, RPT // 16)
    def _(g):
        v = redbuf[0, pl.ds(g * 16, 16)]
        for t in range(1, NS):
            v = v + redbuf[t, pl.ds(g * 16, 16)]
        sumbuf[pl.ds(g * 16, 16)] = v

    pltpu.sync_copy(sumbuf, out_hbm.at[cid, pl.ds(base, RPT)])


SEG = 40
SEGS = 2


@functools.partial(
    pl.kernel,
    out_type=jax.ShapeDtypeStruct((NC, NPAD, D), jnp.float32),
    mesh=_mesh,
    scratch_types=[
        pltpu.VMEM((SEG, KE), jnp.int32),
        pltpu.VMEM((SEG, KE), jnp.int32),
        pltpu.VMEM((KE, D), jnp.float32),
        pltpu.VMEM((KE, D), jnp.float32),
        pltpu.VMEM_SHARED((NPAD, D), jnp.float32),
        pltpu.SemaphoreType.DMA,
        pltpu.SemaphoreType.DMA,
        pltpu.SemaphoreType.DMA,
        pltpu.SemaphoreType.DMA,
        pltpu.SemaphoreType.DMA,
    ],
)
def _agg_kernel(u_hbm, e_hbm, out_hbm, src_t, dst_t,
                rows0, rows1, acc, gsem0, gsem1, ssem0, ssem1, zsem):
    cid = lax.axis_index("c")
    sid = lax.axis_index("s")
    rows = (rows0, rows1)
    gsem = (gsem0, gsem1)
    ssem = (ssem0, ssem1)
    for i in range(KE):
        for j in range(D // 16):
            rows0[i, pl.ds(j * 16, 16)] = jnp.zeros((16,), jnp.float32)
    base = sid * RPT
    zh = [
        pltpu.async_copy(rows0, acc.at[pl.ds(base + i * KE, KE)], zsem)
        for i in range(RPT // KE)
    ]
    zh.append(
        pltpu.async_copy(
            rows0.at[pl.ds(0, RPT - (RPT // KE) * KE)],
            acc.at[pl.ds(base + (RPT // KE) * KE, RPT - (RPT // KE) * KE)],
            zsem,
        )
    )
    for h in zh:
        h.wait()
    plsc.subcore_barrier()

    def pipeline(nseg, row_base):
        for hh in range(nseg):
            hbase = row_base + hh * SEG
            pltpu.sync_copy(e_hbm.at[0, pl.ds(hbase, SEG)], src_t)
            pltpu.sync_copy(e_hbm.at[1, pl.ds(hbase, SEG)], dst_t)
            pend_g = {}
            pend_s = {}
            pend_g[0] = pltpu.async_copy(
                u_hbm.at[src_t.at[0]], rows[0], gsem[0]
            )
            pend_g[1] = pltpu.async_copy(
                u_hbm.at[src_t.at[1]], rows[1], gsem[1]
            )
            for m in range(SEG):
                b = m & 1
                pend_g[m].wait()
                pend_s[m] = pltpu.async_copy(
                    rows[b], acc.at[dst_t.at[m]], ssem[b], add=True
                )
                if m + 2 < SEG:
                    pend_s[m].wait()
                    pend_g[m + 2] = pltpu.async_copy(
                        u_hbm.at[src_t.at[m + 2]], rows[b], gsem[b]
                    )
            pend_s[SEG - 2].wait()
            pend_s[SEG - 1].wait()

    pipeline(SEGS, (cid * NS + sid) * (SEGS * SEG))

    plsc.subcore_barrier()
    pltpu.sync_copy(acc.at[pl.ds(base, RPT)], out_hbm.at[cid, pl.ds(base, RPT)])


BM = 1000


def _prep_body(dp0_ref, dp1_ref, x_ref, w_ref, dis_ref, u_ref):
    deg = 1.0 + dp0_ref[0] + dp1_ref[0]
    dis = lax.rsqrt(deg)
    dis_ref[...] = jnp.broadcast_to(dis, (BM, D))
    h = jnp.dot(x_ref[...], w_ref[...],
                preferred_element_type=jnp.float32,
                precision=lax.Precision.HIGHEST)
    u_ref[...] = dis * h


_prep = pl.pallas_call(
    _prep_body,
    grid=(N // BM,),
    in_specs=[
        pl.BlockSpec((1, BM, 1), lambda i: (0, i, 0)),
        pl.BlockSpec((1, BM, 1), lambda i: (1, i, 0)),
        pl.BlockSpec((BM, D), lambda i: (i, 0)),
        pl.BlockSpec((D, D), lambda i: (0, 0)),
    ],
    out_specs=[
        pl.BlockSpec((BM, D), lambda i: (i, 0)),
        pl.BlockSpec((BM, D), lambda i: (i, 0)),
    ],
    out_shape=[
        jax.ShapeDtypeStruct((N, D), jnp.float32),
        jax.ShapeDtypeStruct((N, D), jnp.float32),
    ],
)


def _layer_body(p0_ref, p1_ref, u_ref, dis_ref, b_ref, w_ref, out_ref):
    dis = dis_ref[...]
    xb = jnp.maximum(
        dis * (p0_ref[0] + p1_ref[0] + u_ref[...]) + b_ref[...], 0.0
    )
    out_ref[...] = dis * jnp.dot(
        xb, w_ref[...],
        preferred_element_type=jnp.float32,
        precision=lax.Precision.HIGHEST,
    )


_layer = pl.pallas_call(
    _layer_body,
    grid=(N // BM,),
    in_specs=[
        pl.BlockSpec((1, BM, D), lambda i: (0, i, 0)),
        pl.BlockSpec((1, BM, D), lambda i: (1, i, 0)),
        pl.BlockSpec((BM, D), lambda i: (i, 0)),
        pl.BlockSpec((BM, D), lambda i: (i, 0)),
        pl.BlockSpec((1, D), lambda i: (0, 0)),
        pl.BlockSpec((D, D), lambda i: (0, 0)),
    ],
    out_specs=pl.BlockSpec((BM, D), lambda i: (i, 0)),
    out_shape=jax.ShapeDtypeStruct((N, D), jnp.float32),
)


BMF = 1000


def _final_body(p0_ref, p1_ref, u_ref, dis_ref, b_ref, out_ref):
    out_ref[...] = (
        dis_ref[...] * (p0_ref[0] + p1_ref[0] + u_ref[...]) + b_ref[...]
    )


_final = pl.pallas_call(
    _final_body,
    grid=(N // BMF,),
    in_specs=[
        pl.BlockSpec((1, BMF, D), lambda i: (0, i, 0)),
        pl.BlockSpec((1, BMF, D), lambda i: (1, i, 0)),
        pl.BlockSpec((BMF, D), lambda i: (i, 0)),
        pl.BlockSpec((BMF, D), lambda i: (i, 0)),
        pl.BlockSpec((1, D), lambda i: (0, 0)),
    ],
    out_specs=pl.BlockSpec((BMF, D), lambda i: (i, 0)),
    out_shape=jax.ShapeDtypeStruct((N, D), jnp.float32),
)


def kernel(x, edge_index, W1, b1, W2, b2, W3, b3, W4, b4):
    e3 = edge_index.astype(jnp.int32).reshape(2, CR, KE)
    dst = edge_index[1].astype(jnp.int32)
    pad = DROWS * 128 - E
    dst_pad = N + (jnp.arange(pad, dtype=jnp.int32) % (NPAD - N))
    dst2 = jnp.concatenate([dst, dst_pad]).reshape(DROWS, 128)

    degp = _deg_kernel(dst2).reshape(NC, NPAD, 1)
    dis, u = _prep(degp, degp, x, W1)

    for (b_prev, w_next) in ((b1, W2), (b2, W3), (b3, W4)):
        p = _agg_kernel(u, e3)
        u = _layer(p, p, u, dis, b_prev.reshape(1, D), w_next)

    p = _agg_kernel(u, e3)
    return _final(p, p, u, dis, b4.reshape(1, D))


# speedup vs baseline: 4.3341x; 1.0011x over previous
import functools

import jax
import jax.numpy as jnp
from jax import lax
from jax.experimental import pallas as pl
from jax.experimental.pallas import tpu as pltpu
from jax.experimental.pallas import tpu_sc as plsc

N = 10000
D = 128
E = 320000

NC = 2
NS = 16
NW = NC * NS
KE = 125
CR = E // KE
CH = CR // NW
NPAD = 10240
RPT = NPAD // NS
DROWS = 2560
DCH = DROWS // NW

_mesh = plsc.VectorSubcoreMesh(
    core_axis_name="c", subcore_axis_name="s", num_cores=NC, num_subcores=NS
)


@functools.partial(
    pl.kernel,
    out_type=jax.ShapeDtypeStruct((NC, NPAD), jnp.float32),
    mesh=_mesh,
    compiler_params=pltpu.CompilerParams(needs_layout_passes=False),
    scratch_types=[
        pltpu.VMEM((40, 128), jnp.int32),
        pltpu.VMEM((NPAD,), jnp.float32),
        pltpu.VMEM((NS, NPAD // NS), jnp.float32),
        pltpu.VMEM((NPAD // NS,), jnp.float32),
        pltpu.VMEM_SHARED((NS, NPAD), jnp.float32),
    ],
)
def _deg_kernel(dst_hbm, out_hbm, dst_t, local, redbuf, sumbuf, slots):
    cid = lax.axis_index("c")
    sid = lax.axis_index("s")
    wid = cid * NS + sid

    @pl.loop(0, NPAD // 16)
    def _(i):
        local[pl.ds(i * 16, 16)] = jnp.zeros((16,), jnp.float32)

    ones = jnp.ones((16,), jnp.float32)
    row0 = wid * DCH
    for seg in range(2):
        pltpu.sync_copy(
            dst_hbm.at[pl.ds(row0 + seg * (DCH // 2), DCH // 2)],
            dst_t.at[pl.ds(0, DCH // 2)],
        )

        @pl.loop(0, DCH // 2)
        def _(r):
            for j in range(8):
                iv = dst_t[r, pl.ds(j * 16, 16)]
                plsc.addupdate_scatter(local, [iv], ones)

    pltpu.sync_copy(local, slots.at[sid])
    plsc.subcore_barrier()
    base = sid * RPT
    pltpu.sync_copy(slots.at[:, pl.ds(base, RPT)], redbuf)

    @pl.loop(0, RPT // 16)
    def _(g):
        v = redbuf[0, pl.ds(g * 16, 16)]
        for t in range(1, NS):
            v = v + redbuf[t, pl.ds(g * 16, 16)]
        sumbuf[pl.ds(g * 16, 16)] = v

    pltpu.sync_copy(sumbuf, out_hbm.at[cid, pl.ds(base, RPT)])


SEG = 40
SEGS = 2


@functools.partial(
    pl.kernel,
    out_type=jax.ShapeDtypeStruct((NC, NPAD, D), jnp.float32),
    mesh=_mesh,
    scratch_types=[
        pltpu.VMEM((SEG, KE), jnp.int32),
        pltpu.VMEM((SEG, KE), jnp.int32),
        pltpu.VMEM((KE, D), jnp.float32),
        pltpu.VMEM((KE, D), jnp.float32),
        pltpu.VMEM_SHARED((NPAD, D), jnp.float32),
        pltpu.SemaphoreType.DMA,
        pltpu.SemaphoreType.DMA,
        pltpu.SemaphoreType.DMA,
        pltpu.SemaphoreType.DMA,
        pltpu.SemaphoreType.DMA,
    ],
)
def _agg_kernel(u_hbm, e_hbm, out_hbm, src_t, dst_t,
                rows0, rows1, acc, gsem0, gsem1, ssem0, ssem1, zsem):
    cid = lax.axis_index("c")
    sid = lax.axis_index("s")
    rows = (rows0, rows1)
    gsem = (gsem0, gsem1)
    ssem = (ssem0, ssem1)
    for i in range(KE):
        for j in range(D // 16):
            rows0[i, pl.ds(j * 16, 16)] = jnp.zeros((16,), jnp.float32)
    base = sid * RPT
    zh = [
        pltpu.async_copy(rows0, acc.at[pl.ds(base + i * KE, KE)], zsem)
        for i in range(RPT // KE)
    ]
    zh.append(
        pltpu.async_copy(
            rows0.at[pl.ds(0, RPT - (RPT // KE) * KE)],
            acc.at[pl.ds(base + (RPT // KE) * KE, RPT - (RPT // KE) * KE)],
            zsem,
        )
    )
    for h in zh:
        h.wait()
    plsc.subcore_barrier()

    def pipeline(nseg, row_base):
        for hh in range(nseg):
            hbase = row_base + hh * SEG
            pltpu.sync_copy(e_hbm.at[0, pl.ds(hbase, SEG)], src_t)
            pltpu.sync_copy(e_hbm.at[1, pl.ds(hbase, SEG)], dst_t)
            pend_g = {}
            pend_s = {}
            pend_g[0] = pltpu.async_copy(
                u_hbm.at[src_t.at[0]], rows[0], gsem[0]
            )
            pend_g[1] = pltpu.async_copy(
                u_hbm.at[src_t.at[1]], rows[1], gsem[1]
            )
            for m in range(SEG):
                b = m & 1
                pend_g[m].wait()
                pend_s[m] = pltpu.async_copy(
                    rows[b], acc.at[dst_t.at[m]], ssem[b], add=True
                )
                if m + 2 < SEG:
                    pend_s[m].wait()
                    pend_g[m + 2] = pltpu.async_copy(
                        u_hbm.at[src_t.at[m + 2]], rows[b], gsem[b]
                    )
            pend_s[SEG - 2].wait()
            pend_s[SEG - 1].wait()

    pipeline(SEGS, (cid * NS + sid) * (SEGS * SEG))

    plsc.subcore_barrier()
    pltpu.sync_copy(acc.at[pl.ds(base, RPT)], out_hbm.at[cid, pl.ds(base, RPT)])


BM = 1000


def _prep_body(dp0_ref, dp1_ref, x_ref, w_ref, dis_ref, u_ref):
    deg = 1.0 + dp0_ref[0] + dp1_ref[0]
    dis = lax.rsqrt(deg)
    dis_ref[...] = dis
    h = jnp.dot(x_ref[...], w_ref[...],
                preferred_element_type=jnp.float32,
                precision=lax.Precision.HIGHEST)
    u_ref[...] = dis * h


_prep = pl.pallas_call(
    _prep_body,
    grid=(N // BM,),
    in_specs=[
        pl.BlockSpec((1, BM, 1), lambda i: (0, i, 0)),
        pl.BlockSpec((1, BM, 1), lambda i: (1, i, 0)),
        pl.BlockSpec((BM, D), lambda i: (i, 0)),
        pl.BlockSpec((D, D), lambda i: (0, 0)),
    ],
    out_specs=[
        pl.BlockSpec((BM, 1), lambda i: (i, 0)),
        pl.BlockSpec((BM, D), lambda i: (i, 0)),
    ],
    out_shape=[
        jax.ShapeDtypeStruct((N, 1), jnp.float32),
        jax.ShapeDtypeStruct((N, D), jnp.float32),
    ],
)


def _layer_body(p0_ref, p1_ref, u_ref, dis_ref, b_ref, w_ref, out_ref):
    dis = dis_ref[...]
    xb = jnp.maximum(
        dis * (p0_ref[0] + p1_ref[0] + u_ref[...]) + b_ref[...], 0.0
    )
    out_ref[...] = dis * jnp.dot(
        xb, w_ref[...],
        preferred_element_type=jnp.float32,
        precision=lax.Precision.HIGHEST,
    )


_layer = pl.pallas_call(
    _layer_body,
    grid=(N // BM,),
    in_specs=[
        pl.BlockSpec((1, BM, D), lambda i: (0, i, 0)),
        pl.BlockSpec((1, BM, D), lambda i: (1, i, 0)),
        pl.BlockSpec((BM, D), lambda i: (i, 0)),
        pl.BlockSpec((BM, 1), lambda i: (i, 0)),
        pl.BlockSpec((1, D), lambda i: (0, 0)),
        pl.BlockSpec((D, D), lambda i: (0, 0)),
    ],
    out_specs=pl.BlockSpec((BM, D), lambda i: (i, 0)),
    out_shape=jax.ShapeDtypeStruct((N, D), jnp.float32),
)


BMF = 1000


def _final_body(p0_ref, p1_ref, u_ref, dis_ref, b_ref, out_ref):
    out_ref[...] = (
        dis_ref[...] * (p0_ref[0] + p1_ref[0] + u_ref[...]) + b_ref[...]
    )


_final = pl.pallas_call(
    _final_body,
    grid=(N // BMF,),
    in_specs=[
        pl.BlockSpec((1, BMF, D), lambda i: (0, i, 0)),
        pl.BlockSpec((1, BMF, D), lambda i: (1, i, 0)),
        pl.BlockSpec((BMF, D), lambda i: (i, 0)),
        pl.BlockSpec((BMF, 1), lambda i: (i, 0)),
        pl.BlockSpec((1, D), lambda i: (0, 0)),
    ],
    out_specs=pl.BlockSpec((BMF, D), lambda i: (i, 0)),
    out_shape=jax.ShapeDtypeStruct((N, D), jnp.float32),
)


def kernel(x, edge_index, W1, b1, W2, b2, W3, b3, W4, b4):
    ei = edge_index.astype(jnp.int32)
    e3 = ei.reshape(2, CR, KE)
    pad = DROWS * 128 - E
    dst_pad = N + (jnp.arange(pad, dtype=jnp.int32) % (NPAD - N))
    dst2 = jnp.concatenate([ei[1], dst_pad]).reshape(DROWS, 128)

    degp = _deg_kernel(dst2).reshape(NC, NPAD, 1)
    dis, u = _prep(degp, degp, x, W1)

    for (b_prev, w_next) in ((b1, W2), (b2, W3), (b3, W4)):
        p = _agg_kernel(u, e3)
        u = _layer(p, p, u, dis, b_prev.reshape(1, D), w_next)

    p = _agg_kernel(u, e3)
    return _final(p, p, u, dis, b4.reshape(1, D))


# speedup vs baseline: 4.4479x; 1.0263x over previous
import functools

import jax
import jax.numpy as jnp
from jax import lax
from jax.experimental import pallas as pl
from jax.experimental.pallas import tpu as pltpu
from jax.experimental.pallas import tpu_sc as plsc

N = 10000
D = 128
E = 320000

NC = 2
NS = 16
NW = NC * NS
KE = 125
CR = E // KE
CH = CR // NW
NPAD = 10240
RPT = NPAD // NS
DROWS = 2560
DCH = DROWS // NW

_mesh = plsc.VectorSubcoreMesh(
    core_axis_name="c", subcore_axis_name="s", num_cores=NC, num_subcores=NS
)


@functools.partial(
    pl.kernel,
    out_type=jax.ShapeDtypeStruct((NC, NPAD), jnp.float32),
    mesh=_mesh,
    compiler_params=pltpu.CompilerParams(needs_layout_passes=False),
    scratch_types=[
        pltpu.VMEM((40, 128), jnp.int32),
        pltpu.VMEM((NPAD,), jnp.float32),
        pltpu.VMEM((NS, NPAD // NS), jnp.float32),
        pltpu.VMEM((NPAD // NS,), jnp.float32),
        pltpu.VMEM_SHARED((NS, NPAD), jnp.float32),
    ],
)
def _deg_kernel(dst_hbm, out_hbm, dst_t, local, redbuf, sumbuf, slots):
    cid = lax.axis_index("c")
    sid = lax.axis_index("s")
    wid = cid * NS + sid

    @pl.loop(0, NPAD // 16)
    def _(i):
        local[pl.ds(i * 16, 16)] = jnp.zeros((16,), jnp.float32)

    ones = jnp.ones((16,), jnp.float32)
    row0 = wid * DCH
    for seg in range(2):
        pltpu.sync_copy(
            dst_hbm.at[pl.ds(row0 + seg * (DCH // 2), DCH // 2)],
            dst_t.at[pl.ds(0, DCH // 2)],
        )

        @pl.loop(0, DCH // 2)
        def _(r):
            for j in range(8):
                iv = dst_t[r, pl.ds(j * 16, 16)]
                plsc.addupdate_scatter(local, [iv], ones)

    pltpu.sync_copy(local, slots.at[sid])
    plsc.subcore_barrier()
    base = sid * RPT
    pltpu.sync_copy(slots.at[:, pl.ds(base, RPT)], redbuf)

    @pl.loop(0, RPT // 16)
    def _(g):
        v = redbuf[0, pl.ds(g * 16, 16)]
        for t in range(1, NS):
            v = v + redbuf[t, pl.ds(g * 16, 16)]
        sumbuf[pl.ds(g * 16, 16)] = v

    pltpu.sync_copy(sumbuf, out_hbm.at[cid, pl.ds(base, RPT)])


SEG = 40
SEGS = 2


@functools.partial(
    pl.kernel,
    out_type=jax.ShapeDtypeStruct((NC, NPAD, D), jnp.float32),
    mesh=_mesh,
    scratch_types=[
        pltpu.VMEM((SEG, KE), jnp.int32),
        pltpu.VMEM((SEG, KE), jnp.int32),
        pltpu.VMEM((KE, D), jnp.float32),
        pltpu.VMEM((KE, D), jnp.float32),
        pltpu.VMEM_SHARED((NPAD, D), jnp.float32),
        pltpu.SemaphoreType.DMA,
        pltpu.SemaphoreType.DMA,
        pltpu.SemaphoreType.DMA,
        pltpu.SemaphoreType.DMA,
        pltpu.SemaphoreType.DMA,
    ],
)
def _agg_kernel(u_hbm, e_hbm, out_hbm, src_t, dst_t,
                rows0, rows1, acc, gsem0, gsem1, ssem0, ssem1, zsem):
    cid = lax.axis_index("c")
    sid = lax.axis_index("s")
    rows = (rows0, rows1)
    gsem = (gsem0, gsem1)
    ssem = (ssem0, ssem1)
    for i in range(KE):
        for j in range(D // 16):
            rows0[i, pl.ds(j * 16, 16)] = jnp.zeros((16,), jnp.float32)
    base = sid * RPT
    zh = [
        pltpu.async_copy(rows0, acc.at[pl.ds(base + i * KE, KE)], zsem)
        for i in range(RPT // KE)
    ]
    zh.append(
        pltpu.async_copy(
            rows0.at[pl.ds(0, RPT - (RPT // KE) * KE)],
            acc.at[pl.ds(base + (RPT // KE) * KE, RPT - (RPT // KE) * KE)],
            zsem,
        )
    )
    for h in zh:
        h.wait()
    plsc.subcore_barrier()

    def pipeline(nseg, row_base):
        for hh in range(nseg):
            hbase = row_base + hh * SEG
            pltpu.sync_copy(e_hbm.at[0, pl.ds(hbase, SEG)], src_t)
            pltpu.sync_copy(e_hbm.at[1, pl.ds(hbase, SEG)], dst_t)
            pend_g = {}
            pend_s = {}
            pend_g[0] = pltpu.async_copy(
                u_hbm.at[src_t.at[0]], rows[0], gsem[0]
            )
            pend_g[1] = pltpu.async_copy(
                u_hbm.at[src_t.at[1]], rows[1], gsem[1]
            )
            for m in range(SEG):
                b = m & 1
                pend_g[m].wait()
                pend_s[m] = pltpu.async_copy(
                    rows[b], acc.at[dst_t.at[m]], ssem[b], add=True
                )
                if m + 2 < SEG:
                    pend_s[m].wait()
                    pend_g[m + 2] = pltpu.async_copy(
                        u_hbm.at[src_t.at[m + 2]], rows[b], gsem[b]
                    )
            pend_s[SEG - 2].wait()
            pend_s[SEG - 1].wait()

    pipeline(SEGS, (cid * NS + sid) * (SEGS * SEG))

    plsc.subcore_barrier()
    pltpu.sync_copy(acc.at[pl.ds(base, RPT)], out_hbm.at[cid, pl.ds(base, RPT)])


BM = 2000


def _prep_body(dp0_ref, dp1_ref, x_ref, w_ref, dis_ref, u_ref):
    deg = 1.0 + dp0_ref[0] + dp1_ref[0]
    dis = lax.rsqrt(deg)
    dis_ref[...] = dis
    h = jnp.dot(x_ref[...], w_ref[...],
                preferred_element_type=jnp.float32,
                precision=lax.Precision.HIGHEST)
    u_ref[...] = dis * h


_prep = pl.pallas_call(
    _prep_body,
    grid=(N // BM,),
    in_specs=[
        pl.BlockSpec((1, BM, 1), lambda i: (0, i, 0)),
        pl.BlockSpec((1, BM, 1), lambda i: (1, i, 0)),
        pl.BlockSpec((BM, D), lambda i: (i, 0)),
        pl.BlockSpec((D, D), lambda i: (0, 0)),
    ],
    out_specs=[
        pl.BlockSpec((BM, 1), lambda i: (i, 0)),
        pl.BlockSpec((BM, D), lambda i: (i, 0)),
    ],
    out_shape=[
        jax.ShapeDtypeStruct((N, 1), jnp.float32),
        jax.ShapeDtypeStruct((N, D), jnp.float32),
    ],
)


def _layer_body(p0_ref, p1_ref, u_ref, dis_ref, b_ref, w_ref, out_ref):
    dis = dis_ref[...]
    xb = jnp.maximum(
        dis * (p0_ref[0] + p1_ref[0] + u_ref[...]) + b_ref[...], 0.0
    )
    out_ref[...] = dis * jnp.dot(
        xb, w_ref[...],
        preferred_element_type=jnp.float32,
        precision=lax.Precision.HIGHEST,
    )


_layer = pl.pallas_call(
    _layer_body,
    grid=(N // BM,),
    in_specs=[
        pl.BlockSpec((1, BM, D), lambda i: (0, i, 0)),
        pl.BlockSpec((1, BM, D), lambda i: (1, i, 0)),
        pl.BlockSpec((BM, D), lambda i: (i, 0)),
        pl.BlockSpec((BM, 1), lambda i: (i, 0)),
        pl.BlockSpec((1, D), lambda i: (0, 0)),
        pl.BlockSpec((D, D), lambda i: (0, 0)),
    ],
    out_specs=pl.BlockSpec((BM, D), lambda i: (i, 0)),
    out_shape=jax.ShapeDtypeStruct((N, D), jnp.float32),
)


BMF = 1000


def _final_body(p0_ref, p1_ref, u_ref, dis_ref, b_ref, out_ref):
    out_ref[...] = (
        dis_ref[...] * (p0_ref[0] + p1_ref[0] + u_ref[...]) + b_ref[...]
    )


_final = pl.pallas_call(
    _final_body,
    grid=(N // BMF,),
    in_specs=[
        pl.BlockSpec((1, BMF, D), lambda i: (0, i, 0)),
        pl.BlockSpec((1, BMF, D), lambda i: (1, i, 0)),
        pl.BlockSpec((BMF, D), lambda i: (i, 0)),
        pl.BlockSpec((BMF, 1), lambda i: (i, 0)),
        pl.BlockSpec((1, D), lambda i: (0, 0)),
    ],
    out_specs=pl.BlockSpec((BMF, D), lambda i: (i, 0)),
    out_shape=jax.ShapeDtypeStruct((N, D), jnp.float32),
)


def kernel(x, edge_index, W1, b1, W2, b2, W3, b3, W4, b4):
    ei = edge_index.astype(jnp.int32)
    e3 = ei.reshape(2, CR, KE)
    pad = DROWS * 128 - E
    dst_pad = N + (jnp.arange(pad, dtype=jnp.int32) % (NPAD - N))
    dst2 = jnp.concatenate([ei[1], dst_pad]).reshape(DROWS, 128)

    degp = _deg_kernel(dst2).reshape(NC, NPAD, 1)
    dis, u = _prep(degp, degp, x, W1)

    for (b_prev, w_next) in ((b1, W2), (b2, W3), (b3, W4)):
        p = _agg_kernel(u, e3)
        u = _layer(p, p, u, dis, b_prev.reshape(1, D), w_next)

    p = _agg_kernel(u, e3)
    return _final(p, p, u, dis, b4.reshape(1, D))


# speedup vs baseline: 4.4756x; 1.0062x over previous
import functools

import jax
import jax.numpy as jnp
from jax import lax
from jax.experimental import pallas as pl
from jax.experimental.pallas import tpu as pltpu
from jax.experimental.pallas import tpu_sc as plsc

N = 10000
D = 128
E = 320000

NC = 2
NS = 16
NW = NC * NS
KE = 125
CR = E // KE
CH = CR // NW
NPAD = 10240
RPT = NPAD // NS
DROWS = 2560
DCH = DROWS // NW

_mesh = plsc.VectorSubcoreMesh(
    core_axis_name="c", subcore_axis_name="s", num_cores=NC, num_subcores=NS
)


@functools.partial(
    pl.kernel,
    out_type=jax.ShapeDtypeStruct((NC, NPAD), jnp.float32),
    mesh=_mesh,
    compiler_params=pltpu.CompilerParams(needs_layout_passes=False),
    scratch_types=[
        pltpu.VMEM((40, 128), jnp.int32),
        pltpu.VMEM((NPAD,), jnp.float32),
        pltpu.VMEM((NS, NPAD // NS), jnp.float32),
        pltpu.VMEM((NPAD // NS,), jnp.float32),
        pltpu.VMEM_SHARED((NS, NPAD), jnp.float32),
    ],
)
def _deg_kernel(dst_hbm, out_hbm, dst_t, local, redbuf, sumbuf, slots):
    cid = lax.axis_index("c")
    sid = lax.axis_index("s")
    wid = cid * NS + sid

    @pl.loop(0, NPAD // 16)
    def _(i):
        local[pl.ds(i * 16, 16)] = jnp.zeros((16,), jnp.float32)

    ones = jnp.ones((16,), jnp.float32)
    row0 = wid * DCH
    for seg in range(2):
        pltpu.sync_copy(
            dst_hbm.at[pl.ds(row0 + seg * (DCH // 2), DCH // 2)],
            dst_t.at[pl.ds(0, DCH // 2)],
        )

        @pl.loop(0, DCH // 2)
        def _(r):
            for j in range(8):
                iv = dst_t[r, pl.ds(j * 16, 16)]
                plsc.addupdate_scatter(local, [iv], ones)

    pltpu.sync_copy(local, slots.at[sid])
    plsc.subcore_barrier()
    base = sid * RPT
    pltpu.sync_copy(slots.at[:, pl.ds(base, RPT)], redbuf)

    @pl.loop(0, RPT // 16)
    def _(g):
        v = redbuf[0, pl.ds(g * 16, 16)]
        for t in range(1, NS):
            v = v + redbuf[t, pl.ds(g * 16, 16)]
        sumbuf[pl.ds(g * 16, 16)] = v

    pltpu.sync_copy(sumbuf, out_hbm.at[cid, pl.ds(base, RPT)])


SEG = 40
SEGS = 2


@functools.partial(
    pl.kernel,
    out_type=jax.ShapeDtypeStruct((NC, NPAD, D), jnp.float32),
    mesh=_mesh,
    scratch_types=[
        pltpu.VMEM((SEG, KE), jnp.int32),
        pltpu.VMEM((SEG, KE), jnp.int32),
        pltpu.VMEM((KE, D), jnp.float32),
        pltpu.VMEM((KE, D), jnp.float32),
        pltpu.VMEM_SHARED((NPAD, D), jnp.float32),
        pltpu.SemaphoreType.DMA,
        pltpu.SemaphoreType.DMA,
        pltpu.SemaphoreType.DMA,
        pltpu.SemaphoreType.DMA,
        pltpu.SemaphoreType.DMA,
    ],
)
def _agg_kernel(u_hbm, e_hbm, out_hbm, src_t, dst_t,
                rows0, rows1, acc, gsem0, gsem1, ssem0, ssem1, zsem):
    cid = lax.axis_index("c")
    sid = lax.axis_index("s")
    rows = (rows0, rows1)
    gsem = (gsem0, gsem1)
    ssem = (ssem0, ssem1)
    for i in range(KE):
        for j in range(D // 16):
            rows0[i, pl.ds(j * 16, 16)] = jnp.zeros((16,), jnp.float32)
    base = sid * RPT
    zh = [
        pltpu.async_copy(rows0, acc.at[pl.ds(base + i * KE, KE)], zsem)
        for i in range(RPT // KE)
    ]
    zh.append(
        pltpu.async_copy(
            rows0.at[pl.ds(0, RPT - (RPT // KE) * KE)],
            acc.at[pl.ds(base + (RPT // KE) * KE, RPT - (RPT // KE) * KE)],
            zsem,
        )
    )
    for h in zh:
        h.wait()
    plsc.subcore_barrier()

    def pipeline(nseg, row_base):
        for hh in range(nseg):
            hbase = row_base + hh * SEG
            pltpu.sync_copy(e_hbm.at[0, pl.ds(hbase, SEG)], src_t)
            pltpu.sync_copy(e_hbm.at[1, pl.ds(hbase, SEG)], dst_t)
            pend_g = {}
            pend_s = {}
            pend_g[0] = pltpu.async_copy(
                u_hbm.at[src_t.at[0]], rows[0], gsem[0]
            )
            pend_g[1] = pltpu.async_copy(
                u_hbm.at[src_t.at[1]], rows[1], gsem[1]
            )
            for m in range(SEG):
                b = m & 1
                pend_g[m].wait()
                pend_s[m] = pltpu.async_copy(
                    rows[b], acc.at[dst_t.at[m]], ssem[b], add=True
                )
                if m + 2 < SEG:
                    pend_s[m].wait()
                    pend_g[m + 2] = pltpu.async_copy(
                        u_hbm.at[src_t.at[m + 2]], rows[b], gsem[b]
                    )
            pend_s[SEG - 2].wait()
            pend_s[SEG - 1].wait()

    pipeline(SEGS, (cid * NS + sid) * (SEGS * SEG))

    plsc.subcore_barrier()
    pltpu.sync_copy(acc.at[pl.ds(base, RPT)], out_hbm.at[cid, pl.ds(base, RPT)])


BM = 2000


def _prep_body(dp0_ref, dp1_ref, x_ref, w_ref, dis_ref, u_ref):
    deg = 1.0 + dp0_ref[0] + dp1_ref[0]
    dis = lax.rsqrt(deg)
    dis_ref[...] = dis
    h = jnp.dot(x_ref[...], w_ref[...],
                preferred_element_type=jnp.float32,
                precision=lax.Precision.HIGHEST)
    u_ref[...] = dis * h


_prep = pl.pallas_call(
    _prep_body,
    grid=(N // BM,),
    in_specs=[
        pl.BlockSpec((1, BM, 1), lambda i: (0, i, 0)),
        pl.BlockSpec((1, BM, 1), lambda i: (1, i, 0)),
        pl.BlockSpec((BM, D), lambda i: (i, 0)),
        pl.BlockSpec((D, D), lambda i: (0, 0)),
    ],
    out_specs=[
        pl.BlockSpec((BM, 1), lambda i: (i, 0)),
        pl.BlockSpec((BM, D), lambda i: (i, 0)),
    ],
    out_shape=[
        jax.ShapeDtypeStruct((N, 1), jnp.float32),
        jax.ShapeDtypeStruct((N, D), jnp.float32),
    ],
)


def _layer_body(p0_ref, p1_ref, u_ref, dis_ref, b_ref, w_ref, out_ref):
    dis = dis_ref[...]
    xb = jnp.maximum(
        dis * (p0_ref[0] + p1_ref[0] + u_ref[...]) + b_ref[...], 0.0
    )
    out_ref[...] = dis * jnp.dot(
        xb, w_ref[...],
        preferred_element_type=jnp.float32,
        precision=lax.Precision.HIGHEST,
    )


_layer = pl.pallas_call(
    _layer_body,
    grid=(N // BM,),
    in_specs=[
        pl.BlockSpec((1, BM, D), lambda i: (0, i, 0)),
        pl.BlockSpec((1, BM, D), lambda i: (1, i, 0)),
        pl.BlockSpec((BM, D), lambda i: (i, 0)),
        pl.BlockSpec((BM, 1), lambda i: (i, 0)),
        pl.BlockSpec((1, D), lambda i: (0, 0)),
        pl.BlockSpec((D, D), lambda i: (0, 0)),
    ],
    out_specs=pl.BlockSpec((BM, D), lambda i: (i, 0)),
    out_shape=jax.ShapeDtypeStruct((N, D), jnp.float32),
)


BMF = 2000


def _final_body(p0_ref, p1_ref, u_ref, dis_ref, b_ref, out_ref):
    out_ref[...] = (
        dis_ref[...] * (p0_ref[0] + p1_ref[0] + u_ref[...]) + b_ref[...]
    )


_final = pl.pallas_call(
    _final_body,
    grid=(N // BMF,),
    in_specs=[
        pl.BlockSpec((1, BMF, D), lambda i: (0, i, 0)),
        pl.BlockSpec((1, BMF, D), lambda i: (1, i, 0)),
        pl.BlockSpec((BMF, D), lambda i: (i, 0)),
        pl.BlockSpec((BMF, 1), lambda i: (i, 0)),
        pl.BlockSpec((1, D), lambda i: (0, 0)),
    ],
    out_specs=pl.BlockSpec((BMF, D), lambda i: (i, 0)),
    out_shape=jax.ShapeDtypeStruct((N, D), jnp.float32),
)


def kernel(x, edge_index, W1, b1, W2, b2, W3, b3, W4, b4):
    ei = edge_index.astype(jnp.int32)
    e3 = ei.reshape(2, CR, KE)
    pad = DROWS * 128 - E
    dst_pad = N + (jnp.arange(pad, dtype=jnp.int32) % (NPAD - N))
    dst2 = jnp.concatenate([ei[1], dst_pad]).reshape(DROWS, 128)

    degp = _deg_kernel(dst2).reshape(NC, NPAD, 1)
    dis, u = _prep(degp, degp, x, W1)

    for (b_prev, w_next) in ((b1, W2), (b2, W3), (b3, W4)):
        p = _agg_kernel(u, e3)
        u = _layer(p, p, u, dis, b_prev.reshape(1, D), w_next)

    p = _agg_kernel(u, e3)
    return _final(p, p, u, dis, b4.reshape(1, D))
